# Initial kernel scaffold; baseline (speedup 1.0000x reference)
import functools

import jax
import jax.numpy as jnp
from jax import lax
from jax.experimental import pallas as pl
from jax.experimental.pallas import tpu as pltpu
from jax.experimental.pallas import tpu_sc as plsc

D = 64
B, L = 4096, 200
N = B * L
NC, NS = 2, 16
NW = NC * NS
ROWS_PER_W = N // NW
W = 128
NBLK = ROWS_PER_W // W
NT = 8

_mesh = plsc.VectorSubcoreMesh(core_axis_name="c", subcore_axis_name="s")


@functools.partial(
    pl.kernel,
    mesh=_mesh,
    out_type=jax.ShapeDtypeStruct((N, D), jnp.float32),
    scratch_types=(
        [pltpu.VMEM((W,), jnp.int32) for _ in range(NT)]
        + [pltpu.VMEM((W, D), jnp.float32) for _ in range(NT)]
        + [pltpu.SemaphoreType.DMA]
    ),
)
def _sc_embed_sum(*refs):
    idx_hbm = refs[:NT]
    tbl_hbm = refs[NT:2 * NT]
    out_hbm = refs[2 * NT]
    idx_v = refs[2 * NT + 1:3 * NT + 1]
    buf_v = refs[3 * NT + 1:4 * NT + 1]
    sem = refs[4 * NT + 1]

    wid = lax.axis_index("s") * NC + lax.axis_index("c")
    base0 = wid * ROWS_PER_W

    @pl.loop(0, NBLK)
    def _blk(blk):
        base = base0 + blk * W

        idx_copies = [
            pltpu.async_copy(idx_hbm[t].at[pl.ds(base, W)], idx_v[t], sem)
            for t in range(NT)
        ]
        for c in idx_copies:
            c.wait()

        gathers = [
            pltpu.async_copy(tbl_hbm[t].at[idx_v[t]], buf_v[t], sem)
            for t in range(NT)
        ]
        for c in gathers:
            c.wait()

        @pl.loop(0, W)
        def _row(r):
            for c in range(0, D, 16):
                v = buf_v[0][r, pl.ds(c, 16)]
                for t in range(1, NT):
                    v = v + buf_v[t][r, pl.ds(c, 16)]
                buf_v[0][r, pl.ds(c, 16)] = v

        pltpu.sync_copy(buf_v[0], out_hbm.at[pl.ds(base, W)])


def kernel(input_ids, pos_ids, position_ids, scrim_ids, start_ids, OffDef,
           token_type_ids, PlayType, T_input_ids, T_pos_ids, T_position_ids,
           T_scrim_ids, T_start_ids, T_OffDef, T_token_type_ids, T_PlayType):
    idxs = [
        x.reshape(-1).astype(jnp.int32)
        for x in (input_ids, pos_ids, position_ids, scrim_ids, start_ids,
                  OffDef, token_type_ids, PlayType)
    ]
    tables = (T_input_ids, T_pos_ids, T_position_ids, T_scrim_ids,
              T_start_ids, T_OffDef, T_token_type_ids, T_PlayType)
    out = _sc_embed_sum(*idxs, *tables)
    return out.reshape(B, L, D)


# speedup vs baseline: 1.5435x; 1.5435x over previous
import functools

import jax
import jax.numpy as jnp
from jax import lax
from jax.experimental import pallas as pl
from jax.experimental.pallas import tpu as pltpu
from jax.experimental.pallas import tpu_sc as plsc

D = 64
B, L = 4096, 200
N = B * L
NC, NS = 2, 16
NW = NC * NS
ROWS_PER_W = N // NW
W = 128
NBLK = ROWS_PER_W // W
NT = 8

_mesh = plsc.VectorSubcoreMesh(core_axis_name="c", subcore_axis_name="s")


@functools.partial(
    pl.kernel,
    mesh=_mesh,
    out_type=jax.ShapeDtypeStruct((N, D), jnp.float32),
    scratch_types=(
        [pltpu.VMEM((W,), jnp.int32) for _ in range(NT)]
        + [pltpu.VMEM((W, D), jnp.float32) for _ in range(NT)]
        + [pltpu.SemaphoreType.DMA]
    ),
    compiler_params=pltpu.CompilerParams(use_tc_tiling_on_sc=False),
)
def _sc_embed_sum(*refs):
    idx_hbm = refs[:NT]
    tbl_hbm = refs[NT:2 * NT]
    out_hbm = refs[2 * NT]
    idx_v = refs[2 * NT + 1:3 * NT + 1]
    buf_v = refs[3 * NT + 1:4 * NT + 1]
    sem = refs[4 * NT + 1]

    wid = lax.axis_index("s") * NC + lax.axis_index("c")
    base0 = wid * ROWS_PER_W

    @pl.loop(0, NBLK)
    def _blk(blk):
        base = base0 + blk * W

        idx_copies = [
            pltpu.async_copy(idx_hbm[t].at[pl.ds(base, W)], idx_v[t], sem)
            for t in range(NT)
        ]
        for c in idx_copies:
            c.wait()

        gathers = [
            pltpu.async_copy(tbl_hbm[t].at[idx_v[t]], buf_v[t], sem)
            for t in range(NT)
        ]
        for c in gathers:
            c.wait()

        @pl.loop(0, W)
        def _row(r):
            for c in range(0, D, 16):
                v = buf_v[0][r, pl.ds(c, 16)]
                for t in range(1, NT):
                    v = v + buf_v[t][r, pl.ds(c, 16)]
                buf_v[0][r, pl.ds(c, 16)] = v

        pltpu.sync_copy(buf_v[0], out_hbm.at[pl.ds(base, W)])


def kernel(input_ids, pos_ids, position_ids, scrim_ids, start_ids, OffDef,
           token_type_ids, PlayType, T_input_ids, T_pos_ids, T_position_ids,
           T_scrim_ids, T_start_ids, T_OffDef, T_token_type_ids, T_PlayType):
    idxs = [
        x.reshape(-1).astype(jnp.int32)
        for x in (input_ids, pos_ids, position_ids, scrim_ids, start_ids,
                  OffDef, token_type_ids, PlayType)
    ]
    tables = (T_input_ids, T_pos_ids, T_position_ids, T_scrim_ids,
              T_start_ids, T_OffDef, T_token_type_ids, T_PlayType)
    out = _sc_embed_sum(*idxs, *tables)
    return out.reshape(B, L, D)


# speedup vs baseline: 12.2918x; 7.9636x over previous
import functools

import jax
import jax.numpy as jnp
from jax import lax
from jax.experimental import pallas as pl
from jax.experimental.pallas import tpu as pltpu
from jax.experimental.pallas import tpu_sc as plsc

D = 64
B, L = 4096, 200
N = B * L
NC, NS = 2, 16
NW = NC * NS
ROWS_PER_W = N // NW
W = 128
NBLK = ROWS_PER_W // W
NBLK_TOT = N // W
NIDX = 8
NSTR = 4

V_OFF, V_TT, V_PT, V_POS2 = 4, 8, 32, 32
V_SCRIM, V_START = 128, 128
VG1 = V_OFF * V_TT * V_PT * V_POS2
VG2 = V_SCRIM * V_START


def _build_tables_body(t_off, t_tt, t_pt, t_pos2, t_scrim, t_start, g1, g2):
    a = t_off[...].reshape(V_OFF, 1, D) + t_tt[...].reshape(1, V_TT, D)
    a = a.reshape(V_OFF * V_TT, D)
    bb = t_pt[...].reshape(V_PT, 1, D) + t_pos2[...].reshape(1, V_POS2, D)
    bb = bb.reshape(V_PT * V_POS2, D)
    g1[...] = (a.reshape(V_OFF * V_TT, 1, D)
               + bb.reshape(1, V_PT * V_POS2, D)).reshape(VG1, D)
    g2[...] = (t_scrim[...].reshape(V_SCRIM, 1, D)
               + t_start[...].reshape(1, V_START, D)).reshape(VG2, D)


_build_tables = pl.pallas_call(
    _build_tables_body,
    out_shape=[jax.ShapeDtypeStruct((VG1, D), jnp.float32),
               jax.ShapeDtypeStruct((VG2, D), jnp.float32)],
)

_mesh = plsc.VectorSubcoreMesh(core_axis_name="c", subcore_axis_name="s")


@functools.partial(
    pl.kernel,
    mesh=_mesh,
    out_type=jax.ShapeDtypeStruct((N, D), jnp.float32),
    scratch_types=(
        [pltpu.VMEM((NIDX, W), jnp.int32) for _ in range(2)]
        + [pltpu.VMEM((W, D), jnp.float32) for _ in range(2 * NSTR)]
        + [pltpu.VMEM((W, D), jnp.float32) for _ in range(2)]
        + [pltpu.SemaphoreType.DMA] * 6
    ),
    compiler_params=pltpu.CompilerParams(use_tc_tiling_on_sc=False),
)
def _sc_embed4(idx_hbm, big_hbm, pos_hbm, g1_hbm, g2_hbm, out_hbm,
               idx0, idx1, b00, b01, b02, b03, b10, b11, b12, b13,
               ob0, ob1, si0, si1, sg0, sg1, so0, so1):
    idx_v = (idx0, idx1)
    bufs = ((b00, b01, b02, b03), (b10, b11, b12, b13))
    outb = (ob0, ob1)
    sem_idx = (si0, si1)
    sem_g = (sg0, sg1)
    sem_out = (so0, so1)

    wid = lax.axis_index("s") * NC + lax.axis_index("c")
    g0 = wid * NBLK
    tables = (big_hbm, pos_hbm, g1_hbm, g2_hbm)

    def fire_idx(j, s):
        pltpu.async_copy(idx_hbm.at[g0 + j], idx_v[s], sem_idx[s])

    def wait_idx(j, s):
        pltpu.make_async_copy(idx_hbm.at[g0 + j], idx_v[s], sem_idx[s]).wait()

    def fuse(s):
        iv = idx_v[s]

        @pl.loop(0, W, step=16)
        def _(k):
            sl = pl.ds(k, 16)
            r2 = iv[2, sl]
            r3 = iv[3, sl]
            r4 = iv[4, sl]
            r5 = iv[5, sl]
            r6 = iv[6, sl]
            r7 = iv[7, sl]
            iv[2, sl] = ((r5 * V_TT + r6) * V_PT + r7) * V_POS2 + r2
            iv[3, sl] = r3 * V_START + r4

    def fire_gathers(s):
        for t in range(NSTR):
            pltpu.async_copy(tables[t].at[idx_v[s].at[t]], bufs[s][t], sem_g[s])

    def wait_gathers(s):
        for t in range(NSTR):
            pltpu.make_async_copy(
                tables[t].at[idx_v[s].at[t]], bufs[s][t], sem_g[s]).wait()

    def adds(s):
        b0, b1, b2, b3 = bufs[s]
        ob = outb[s]

        @pl.loop(0, W)
        def _(r):
            for c in range(0, D, 16):
                sl = pl.ds(c, 16)
                ob[r, sl] = b0[r, sl] + b1[r, sl] + b2[r, sl] + b3[r, sl]

    def fire_out(j, s):
        pltpu.async_copy(outb[s], out_hbm.at[pl.ds((g0 + j) * W, W)],
                         sem_out[s])

    def wait_out(j, s):
        pltpu.make_async_copy(outb[s], out_hbm.at[pl.ds((g0 + j) * W, W)],
                              sem_out[s]).wait()

    fire_idx(0, 0)
    fire_idx(1, 1)
    wait_idx(0, 0)
    fuse(0)
    fire_gathers(0)

    @pl.loop(0, NBLK, step=2)
    def _outer(jj):
        for s in (0, 1):
            j = jj + s
            o = s ^ 1

            @pl.when(j + 1 < NBLK)
            def _():
                wait_idx(j + 1, o)
                fuse(o)
                fire_gathers(o)

            wait_gathers(s)

            @pl.when(j >= 2)
            def _():
                wait_out(j - 2, s)

            adds(s)
            fire_out(j, s)

            @pl.when(j + 2 < NBLK)
            def _():
                fire_idx(j + 2, s)

    wait_out(NBLK - 2, 0)
    wait_out(NBLK - 1, 1)


def kernel(input_ids, pos_ids, position_ids, scrim_ids, start_ids, OffDef,
           token_type_ids, PlayType, T_input_ids, T_pos_ids, T_position_ids,
           T_scrim_ids, T_start_ids, T_OffDef, T_token_type_ids, T_PlayType):
    idx = jnp.stack([
        x.reshape(-1).astype(jnp.int32)
        for x in (input_ids, pos_ids, position_ids, scrim_ids, start_ids,
                  OffDef, token_type_ids, PlayType)
    ])
    idx3 = idx.reshape(NIDX, NBLK_TOT, W).transpose(1, 0, 2)
    g1, g2 = _build_tables(T_OffDef, T_token_type_ids, T_PlayType,
                           T_position_ids, T_scrim_ids, T_start_ids)
    out = _sc_embed4(idx3, T_input_ids, T_pos_ids, g1, g2)
    return out.reshape(B, L, D)


# speedup vs baseline: 12.5448x; 1.0206x over previous
import functools

import jax
import jax.numpy as jnp
from jax import lax
from jax.experimental import pallas as pl
from jax.experimental.pallas import tpu as pltpu
from jax.experimental.pallas import tpu_sc as plsc

D = 64
B, L = 4096, 200
N = B * L
NC, NS = 2, 16
NW = NC * NS
W = 200
WPAD = 208
NBLK = B // NW
NIDX = 8
NSTR = 4

V_OFF, V_TT, V_PT, V_POS2 = 4, 8, 32, 32
V_SCRIM, V_START = 128, 128
VG1 = V_OFF * V_TT * V_PT * V_POS2
VG2 = V_SCRIM * V_START


def _build_tables_body(t_off, t_tt, t_pt, t_pos2, t_scrim, t_start, g1, g2):
    a = t_off[...].reshape(V_OFF, 1, D) + t_tt[...].reshape(1, V_TT, D)
    a = a.reshape(V_OFF * V_TT, D)
    bb = t_pt[...].reshape(V_PT, 1, D) + t_pos2[...].reshape(1, V_POS2, D)
    bb = bb.reshape(V_PT * V_POS2, D)
    g1[...] = (a.reshape(V_OFF * V_TT, 1, D)
               + bb.reshape(1, V_PT * V_POS2, D)).reshape(VG1, D)
    g2[...] = (t_scrim[...].reshape(V_SCRIM, 1, D)
               + t_start[...].reshape(1, V_START, D)).reshape(VG2, D)


_build_tables = pl.pallas_call(
    _build_tables_body,
    out_shape=[jax.ShapeDtypeStruct((VG1, D), jnp.float32),
               jax.ShapeDtypeStruct((VG2, D), jnp.float32)],
)

_mesh = plsc.VectorSubcoreMesh(core_axis_name="c", subcore_axis_name="s")


@functools.partial(
    pl.kernel,
    mesh=_mesh,
    out_type=jax.ShapeDtypeStruct((B, L, D), jnp.float32),
    scratch_types=(
        [pltpu.VMEM((NIDX, WPAD), jnp.int32) for _ in range(2)]
        + [pltpu.VMEM((W, D), jnp.float32) for _ in range(2 * (NSTR - 1))]
        + [pltpu.VMEM((W, D), jnp.float32) for _ in range(2)]
        + [pltpu.SemaphoreType.DMA] * 6
    ),
    compiler_params=pltpu.CompilerParams(use_tc_tiling_on_sc=False),
)
def _sc_embed4(i0, i1, i2, i3, i4, i5, i6, i7,
               big_hbm, pos_hbm, g1_hbm, g2_hbm, out_hbm,
               idx0, idx1, b01, b02, b03, b11, b12, b13,
               ob0, ob1, si0, si1, sg0, sg1, so0, so1):
    idx_hbm = (i0, i1, i2, i3, i4, i5, i6, i7)
    idx_v = (idx0, idx1)
    outb = (ob0, ob1)
    bufs = ((ob0, b01, b02, b03), (ob1, b11, b12, b13))
    sem_idx = (si0, si1)
    sem_g = (sg0, sg1)
    sem_out = (so0, so1)

    wid = lax.axis_index("s") * NC + lax.axis_index("c")
    g0 = wid * NBLK
    tables = (big_hbm, pos_hbm, g1_hbm, g2_hbm)

    def fire_idx(j, s):
        for t in range(NIDX):
            pltpu.async_copy(idx_hbm[t].at[pl.ds((g0 + j) * W, W)],
                             idx_v[s].at[t, pl.ds(0, W)], sem_idx[s])

    def wait_idx(j, s):
        for t in range(NIDX):
            pltpu.make_async_copy(idx_hbm[t].at[pl.ds((g0 + j) * W, W)],
                                  idx_v[s].at[t, pl.ds(0, W)],
                                  sem_idx[s]).wait()

    def fuse(s):
        iv = idx_v[s]
        for k in range(0, WPAD, 16):
            sl = pl.ds(k, 16)
            r2 = iv[2, sl]
            r3 = iv[3, sl]
            r4 = iv[4, sl]
            r5 = iv[5, sl]
            r6 = iv[6, sl]
            r7 = iv[7, sl]
            iv[2, sl] = ((r5 * V_TT + r6) * V_PT + r7) * V_POS2 + r2
            iv[3, sl] = r3 * V_START + r4

    def fire_gathers(s):
        for t in range(NSTR):
            pltpu.async_copy(tables[t].at[idx_v[s].at[t, pl.ds(0, W)]],
                             bufs[s][t], sem_g[s])

    def wait_gathers(s):
        for t in range(NSTR):
            pltpu.make_async_copy(tables[t].at[idx_v[s].at[t, pl.ds(0, W)]],
                                  bufs[s][t], sem_g[s]).wait()

    def adds(s):
        ob, b1, b2, b3 = bufs[s]

        @pl.loop(0, W)
        def _(r):
            for c in range(0, D, 16):
                sl = pl.ds(c, 16)
                ob[r, sl] = ob[r, sl] + b1[r, sl] + b2[r, sl] + b3[r, sl]

    def fire_out(j, s):
        pltpu.async_copy(outb[s], out_hbm.at[g0 + j], sem_out[s])

    def wait_out(j, s):
        pltpu.make_async_copy(outb[s], out_hbm.at[g0 + j], sem_out[s]).wait()

    fire_idx(0, 0)
    fire_idx(1, 1)
    wait_idx(0, 0)
    fuse(0)
    fire_gathers(0)

    @pl.loop(0, NBLK, step=2)
    def _outer(jj):
        for s in (0, 1):
            j = jj + s
            o = s ^ 1

            @pl.when(j >= 1)
            def _():
                wait_out(j - 1, o)

            @pl.when(j + 1 < NBLK)
            def _():
                wait_idx(j + 1, o)
                fuse(o)
                fire_gathers(o)

            wait_gathers(s)
            adds(s)
            fire_out(j, s)

            @pl.when(j + 2 < NBLK)
            def _():
                fire_idx(j + 2, s)

    wait_out(NBLK - 1, 1)


def kernel(input_ids, pos_ids, position_ids, scrim_ids, start_ids, OffDef,
           token_type_ids, PlayType, T_input_ids, T_pos_ids, T_position_ids,
           T_scrim_ids, T_start_ids, T_OffDef, T_token_type_ids, T_PlayType):
    idxs = [
        x.astype(jnp.int32).reshape(-1)
        for x in (input_ids, pos_ids, position_ids, scrim_ids, start_ids,
                  OffDef, token_type_ids, PlayType)
    ]
    g1, g2 = _build_tables(T_OffDef, T_token_type_ids, T_PlayType,
                           T_position_ids, T_scrim_ids, T_start_ids)
    return _sc_embed4(*idxs, T_input_ids, T_pos_ids, g1, g2)


# speedup vs baseline: 15.3786x; 1.2259x over previous
import functools

import jax
import jax.numpy as jnp
from jax import lax
from jax.experimental import pallas as pl
from jax.experimental.pallas import tpu as pltpu
from jax.experimental.pallas import tpu_sc as plsc

D = 64
B, L = 4096, 200
N = B * L
NC, NS = 2, 16
NW = NC * NS
W = 200
WPAD = 208
NBLK = B // NW
NIDX = 8
NSTR = 4

V_OFF, V_TT, V_PT, V_POS2 = 4, 8, 32, 32
V_SCRIM, V_START = 128, 128
VG1 = V_OFF * V_TT * V_PT * V_POS2
VG2 = V_SCRIM * V_START


def _build_tables_body(t_off, t_tt, t_pt, t_pos2, t_scrim, t_start, g1, g2):
    a = t_off[...].reshape(V_OFF, 1, D) + t_tt[...].reshape(1, V_TT, D)
    a = a.reshape(V_OFF * V_TT, D)
    bb = t_pt[...].reshape(V_PT, 1, D) + t_pos2[...].reshape(1, V_POS2, D)
    bb = bb.reshape(V_PT * V_POS2, D)
    g1[...] = (a.reshape(V_OFF * V_TT, 1, D)
               + bb.reshape(1, V_PT * V_POS2, D)).reshape(VG1, D)
    g2[...] = (t_scrim[...].reshape(V_SCRIM, 1, D)
               + t_start[...].reshape(1, V_START, D)).reshape(VG2, D)


_build_tables = pl.pallas_call(
    _build_tables_body,
    out_shape=[jax.ShapeDtypeStruct((VG1, D), jnp.float32),
               jax.ShapeDtypeStruct((VG2, D), jnp.float32)],
)

_mesh = plsc.VectorSubcoreMesh(core_axis_name="c", subcore_axis_name="s")


@functools.partial(
    pl.kernel,
    mesh=_mesh,
    out_type=jax.ShapeDtypeStruct((N, 2 * D), jnp.float32),
    scratch_types=(
        [pltpu.VMEM((NIDX, WPAD), jnp.int32) for _ in range(2)]
        + [pltpu.VMEM((W, D), jnp.float32) for _ in range(2 * (NSTR - 1))]
        + [pltpu.VMEM((W, D), jnp.float32) for _ in range(2)]
        + [pltpu.SemaphoreType.DMA] * 6
    ),
    compiler_params=pltpu.CompilerParams(use_tc_tiling_on_sc=False),
)
def _sc_embed4(i0, i1, i2, i3, i4, i5, i6, i7,
               big_hbm, pos_hbm, g1_hbm, g2_hbm, out_hbm,
               idx0, idx1, b01, b02, b03, b11, b12, b13,
               ob0, ob1, si0, si1, sg0, sg1, so0, so1):
    idx_hbm = (i0, i1, i2, i3, i4, i5, i6, i7)
    idx_v = (idx0, idx1)
    outb = (ob0, ob1)
    bufs = ((ob0, b01, b02, b03), (ob1, b11, b12, b13))
    sem_idx = (si0, si1)
    sem_g = (sg0, sg1)
    sem_out = (so0, so1)

    wid = lax.axis_index("s") * NC + lax.axis_index("c")
    g0 = wid * NBLK
    tables = (big_hbm, pos_hbm, g1_hbm, g2_hbm)

    def fire_idx(j, s):
        for t in range(NIDX):
            pltpu.async_copy(idx_hbm[t].at[pl.ds((g0 + j) * W, W)],
                             idx_v[s].at[t, pl.ds(0, W)], sem_idx[s])

    def wait_idx(j, s):
        for t in range(NIDX):
            pltpu.make_async_copy(idx_hbm[t].at[pl.ds((g0 + j) * W, W)],
                                  idx_v[s].at[t, pl.ds(0, W)],
                                  sem_idx[s]).wait()

    def fuse(s):
        iv = idx_v[s]
        for k in range(0, WPAD, 16):
            sl = pl.ds(k, 16)
            r2 = iv[2, sl]
            r3 = iv[3, sl]
            r4 = iv[4, sl]
            r5 = iv[5, sl]
            r6 = iv[6, sl]
            r7 = iv[7, sl]
            iv[2, sl] = ((r5 * V_TT + r6) * V_PT + r7) * V_POS2 + r2
            iv[3, sl] = r3 * V_START + r4

    def fire_gathers(s):
        for t in range(NSTR):
            pltpu.async_copy(tables[t].at[idx_v[s].at[t, pl.ds(0, W)]],
                             bufs[s][t], sem_g[s])

    def wait_gathers(s):
        for t in range(NSTR):
            pltpu.make_async_copy(tables[t].at[idx_v[s].at[t, pl.ds(0, W)]],
                                  bufs[s][t], sem_g[s]).wait()

    def adds(s):
        ob, b1, b2, b3 = bufs[s]

        @pl.loop(0, W)
        def _(r):
            for c in range(0, D, 16):
                sl = pl.ds(c, 16)
                ob[r, sl] = ob[r, sl] + b1[r, sl] + b2[r, sl] + b3[r, sl]

    def fire_out(j, s):
        pltpu.async_copy(outb[s],
                         out_hbm.at[pl.ds((g0 + j) * W, W), pl.ds(0, D)],
                         sem_out[s])

    def wait_out(j, s):
        pltpu.make_async_copy(outb[s],
                              out_hbm.at[pl.ds((g0 + j) * W, W), pl.ds(0, D)],
                              sem_out[s]).wait()

    fire_idx(0, 0)
    fire_idx(1, 1)
    wait_idx(0, 0)
    fuse(0)
    fire_gathers(0)

    @pl.loop(0, NBLK, step=2)
    def _outer(jj):
        for s in (0, 1):
            j = jj + s
            o = s ^ 1

            @pl.when(j >= 1)
            def _():
                wait_out(j - 1, o)

            @pl.when(j + 1 < NBLK)
            def _():
                wait_idx(j + 1, o)
                fuse(o)
                fire_gathers(o)

            wait_gathers(s)
            adds(s)
            fire_out(j, s)

            @pl.when(j + 2 < NBLK)
            def _():
                fire_idx(j + 2, s)

    wait_out(NBLK - 1, 1)


def kernel(input_ids, pos_ids, position_ids, scrim_ids, start_ids, OffDef,
           token_type_ids, PlayType, T_input_ids, T_pos_ids, T_position_ids,
           T_scrim_ids, T_start_ids, T_OffDef, T_token_type_ids, T_PlayType):
    idxs = [
        x.astype(jnp.int32).reshape(-1)
        for x in (input_ids, pos_ids, position_ids, scrim_ids, start_ids,
                  OffDef, token_type_ids, PlayType)
    ]
    g1, g2 = _build_tables(T_OffDef, T_token_type_ids, T_PlayType,
                           T_position_ids, T_scrim_ids, T_start_ids)
    out = _sc_embed4(*idxs, T_input_ids, T_pos_ids, g1, g2)
    return out.reshape(B, L, 2 * D)[:, :, :D]


# speedup vs baseline: 16.1160x; 1.0480x over previous
import functools

import jax
import jax.numpy as jnp
from jax import lax
from jax.experimental import pallas as pl
from jax.experimental.pallas import tpu as pltpu
from jax.experimental.pallas import tpu_sc as plsc

D = 64
B, L = 4096, 200
N = B * L
NC, NS = 2, 16
NW = NC * NS
W = 200
WPAD = 208
NBLK = B // NW
NIDX = 8
NSTR = 4

V_OFF, V_TT, V_PT, V_POS2 = 4, 8, 32, 32
V_SCRIM, V_START = 128, 128
VG1 = V_OFF * V_TT * V_PT * V_POS2
VG2 = V_SCRIM * V_START


def _build_tables_body(t_off, t_tt, t_pt, t_pos2, t_scrim, t_start, g1, g2):
    a = t_off[...].reshape(V_OFF, 1, D) + t_tt[...].reshape(1, V_TT, D)
    a = a.reshape(V_OFF * V_TT, D)
    bb = t_pt[...].reshape(V_PT, 1, D) + t_pos2[...].reshape(1, V_POS2, D)
    bb = bb.reshape(V_PT * V_POS2, D)
    g1[:, 0:D] = (a.reshape(V_OFF * V_TT, 1, D)
                  + bb.reshape(1, V_PT * V_POS2, D)).reshape(VG1, D)
    g2[:, 0:D] = (t_scrim[...].reshape(V_SCRIM, 1, D)
                  + t_start[...].reshape(1, V_START, D)).reshape(VG2, D)


_build_tables = pl.pallas_call(
    _build_tables_body,
    out_shape=[jax.ShapeDtypeStruct((VG1, 2 * D), jnp.float32),
               jax.ShapeDtypeStruct((VG2, 2 * D), jnp.float32)],
)

VOC_BIG = 1000000
TT_BLK = 2048


def _transpose_body(x_ref, o_ref):
    o_ref[:, 0:D] = x_ref[...].T


_transpose_table = pl.pallas_call(
    _transpose_body,
    grid=((VOC_BIG + TT_BLK - 1) // TT_BLK,),
    in_specs=[pl.BlockSpec((D, TT_BLK), lambda i: (0, i))],
    out_specs=pl.BlockSpec((TT_BLK, 2 * D), lambda i: (i, 0)),
    out_shape=jax.ShapeDtypeStruct((VOC_BIG, 2 * D), jnp.float32),
)

_mesh = plsc.VectorSubcoreMesh(core_axis_name="c", subcore_axis_name="s")


@functools.partial(
    pl.kernel,
    mesh=_mesh,
    out_type=jax.ShapeDtypeStruct((N, 2 * D), jnp.float32),
    scratch_types=(
        [pltpu.VMEM((NIDX, WPAD), jnp.int32) for _ in range(2)]
        + [pltpu.VMEM((W, D), jnp.float32) for _ in range(2 * (NSTR - 1))]
        + [pltpu.VMEM((W, D), jnp.float32) for _ in range(2)]
        + [pltpu.SemaphoreType.DMA] * 6
    ),
    compiler_params=pltpu.CompilerParams(use_tc_tiling_on_sc=False),
)
def _sc_embed4(i0, i1, i2, i3, i4, i5, i6, i7,
               big_hbm, pos_hbm, g1_hbm, g2_hbm, out_hbm,
               idx0, idx1, b01, b02, b03, b11, b12, b13,
               ob0, ob1, si0, si1, sg0, sg1, so0, so1):
    idx_hbm = (i0, i1, i2, i3, i4, i5, i6, i7)
    idx_v = (idx0, idx1)
    outb = (ob0, ob1)
    bufs = ((ob0, b01, b02, b03), (ob1, b11, b12, b13))
    sem_idx = (si0, si1)
    sem_g = (sg0, sg1)
    sem_out = (so0, so1)

    wid = lax.axis_index("s") * NC + lax.axis_index("c")
    g0 = wid * NBLK
    tables = (big_hbm, pos_hbm, g1_hbm, g2_hbm)

    def fire_idx(j, s):
        for t in range(NIDX):
            pltpu.async_copy(idx_hbm[t].at[pl.ds((g0 + j) * W, W)],
                             idx_v[s].at[t, pl.ds(0, W)], sem_idx[s])

    def wait_idx(j, s):
        for t in range(NIDX):
            pltpu.make_async_copy(idx_hbm[t].at[pl.ds((g0 + j) * W, W)],
                                  idx_v[s].at[t, pl.ds(0, W)],
                                  sem_idx[s]).wait()

    def fuse(s):
        iv = idx_v[s]
        for k in range(0, WPAD, 16):
            sl = pl.ds(k, 16)
            r0 = iv[0, sl]
            r2 = iv[2, sl]
            r3 = iv[3, sl]
            r4 = iv[4, sl]
            r5 = iv[5, sl]
            r6 = iv[6, sl]
            r7 = iv[7, sl]
            iv[0, sl] = r0 * 2
            iv[2, sl] = (((r5 * V_TT + r6) * V_PT + r7) * V_POS2 + r2) * 2
            iv[3, sl] = (r3 * V_START + r4) * 2

    def fire_gathers(s):
        for t in range(NSTR):
            pltpu.async_copy(tables[t].at[idx_v[s].at[t, pl.ds(0, W)]],
                             bufs[s][t], sem_g[s])

    def wait_gathers(s):
        for t in range(NSTR):
            pltpu.make_async_copy(tables[t].at[idx_v[s].at[t, pl.ds(0, W)]],
                                  bufs[s][t], sem_g[s]).wait()

    def adds(s):
        ob, b1, b2, b3 = bufs[s]

        @pl.loop(0, W)
        def _(r):
            for c in range(0, D, 16):
                sl = pl.ds(c, 16)
                ob[r, sl] = ob[r, sl] + b1[r, sl] + b2[r, sl] + b3[r, sl]

    def fire_out(j, s):
        pltpu.async_copy(outb[s],
                         out_hbm.at[pl.ds((g0 + j) * W, W), pl.ds(0, D)],
                         sem_out[s])

    def wait_out(j, s):
        pltpu.make_async_copy(outb[s],
                              out_hbm.at[pl.ds((g0 + j) * W, W), pl.ds(0, D)],
                              sem_out[s]).wait()

    fire_idx(0, 0)
    fire_idx(1, 1)
    wait_idx(0, 0)
    fuse(0)
    fire_gathers(0)

    @pl.loop(0, NBLK, step=2)
    def _outer(jj):
        for s in (0, 1):
            j = jj + s
            o = s ^ 1

            @pl.when(j >= 1)
            def _():
                wait_out(j - 1, o)

            @pl.when(j + 1 < NBLK)
            def _():
                wait_idx(j + 1, o)
                fuse(o)
                fire_gathers(o)

            wait_gathers(s)
            adds(s)
            fire_out(j, s)

            @pl.when(j + 2 < NBLK)
            def _():
                fire_idx(j + 2, s)

    wait_out(NBLK - 1, 1)


def kernel(input_ids, pos_ids, position_ids, scrim_ids, start_ids, OffDef,
           token_type_ids, PlayType, T_input_ids, T_pos_ids, T_position_ids,
           T_scrim_ids, T_start_ids, T_OffDef, T_token_type_ids, T_PlayType):
    idxs = [
        x.astype(jnp.int32).reshape(-1)
        for x in (input_ids, pos_ids, position_ids, scrim_ids, start_ids,
                  OffDef, token_type_ids, PlayType)
    ]
    g1, g2 = _build_tables(T_OffDef, T_token_type_ids, T_PlayType,
                           T_position_ids, T_scrim_ids, T_start_ids)
    big = _transpose_table(T_input_ids.T).reshape(2 * VOC_BIG, D)
    out = _sc_embed4(*idxs, big, T_pos_ids,
                     g1.reshape(2 * VG1, D), g2.reshape(2 * VG2, D))
    return out.reshape(B, L, 2 * D)[:, :, :D]
